# baseline (speedup 1.0000x reference)
import dataclasses
import functools
import jax
import jax.numpy as jnp
from jax import lax
from jax.experimental import pallas as pl
from jax.experimental.pallas import tpu as pltpu
from jax.experimental.pallas import tpu_sc as plsc

_N = 20000
_L = 16
_NS = 16
_ROWS = 80
_PER = _ROWS * _L
_NP = _NS * _PER
_MAX_DET = 100
_SCORE_THRESH = 0.05
_NMS_THRESH = 0.5
_BIG_I = 1 << 30


def _nms_body(s_hbm, x1_hbm, y1_hbm, x2_hbm, y2_hbm, out_hbm,
              sw_v, so_v, x1_v, y1_v, x2_v, y2_v, ar_v,
              pub_v, shared_v, cand_v, out_v):
    c = lax.axis_index("c")
    s = lax.axis_index("s")
    base = s * _PER
    iot = lax.iota(jnp.int32, _L)
    ninf = jnp.float32(-jnp.inf)

    pltpu.sync_copy(s_hbm.at[s], so_v)
    pltpu.sync_copy(x1_hbm.at[s], x1_v)
    pltpu.sync_copy(y1_hbm.at[s], y1_v)
    pltpu.sync_copy(x2_hbm.at[s], x2_v)
    pltpu.sync_copy(y2_hbm.at[s], y2_v)

    def publish(bv, bi, buf_row):
        m_v = jnp.broadcast_to(jnp.max(bv), (_L,))
        li_v = jnp.broadcast_to(
            jnp.min(jnp.where(bv == m_v, bi, jnp.int32(_BIG_I))), (_L,))
        loc_v = li_v - base
        rowv = lax.shift_right_arithmetic(loc_v, 4)
        offv = lax.bitwise_and(loc_v, 15)
        gx1 = plsc.load_gather(x1_v, [rowv, offv])
        gy1 = plsc.load_gather(y1_v, [rowv, offv])
        gx2 = plsc.load_gather(x2_v, [rowv, offv])
        gy2 = plsc.load_gather(y2_v, [rowv, offv])
        gar = plsc.load_gather(ar_v, [rowv, offv])
        gso = plsc.load_gather(so_v, [rowv, offv])
        data = jnp.where(iot == 0, gx1,
               jnp.where(iot == 1, gy1,
               jnp.where(iot == 2, gx2,
               jnp.where(iot == 3, gy2,
               jnp.where(iot == 4, gso, gar)))))
        pub_v[0, 0:_L] = m_v
        pub_v[0, _L:2 * _L] = plsc.bitcast(li_v, jnp.float32)
        pub_v[0, 2 * _L:3 * _L] = data
        pltpu.sync_copy(pub_v, shared_v.at[pl.ds(buf_row + s, 1)])

    def prep(r, carry):
        bv, bi = carry
        v = so_v[r]
        sw = jnp.where(v > _SCORE_THRESH, v, ninf)
        sw_v[r] = sw
        ar_v[r] = (x2_v[r] - x1_v[r]) * (y2_v[r] - y1_v[r])
        cond = sw > bv
        idxv = iot + (base + r * _L)
        return jnp.where(cond, sw, bv), jnp.where(cond, idxv, bi)

    bv0 = jnp.full((_L,), ninf, jnp.float32)
    bi0 = jnp.full((_L,), base, jnp.int32)
    bv, bi = lax.fori_loop(0, _ROWS, prep, (bv0, bi0))
    publish(bv, bi, jnp.int32(0))

    zero_v = jnp.zeros((_L,), jnp.int32)
    one_v = jnp.full((_L,), _L, jnp.int32)

    @pl.loop(0, _MAX_DET)
    def _round(it):
        roff = lax.bitwise_and(it, 1) * _NS
        plsc.subcore_barrier()
        pltpu.sync_copy(shared_v.at[pl.ds(roff, _NS)], cand_v)

        val_vec = plsc.load_gather(cand_v, [iot, zero_v])
        idx_vec = plsc.bitcast(plsc.load_gather(cand_v, [iot, one_v]),
                               jnp.int32)
        wv = jnp.broadcast_to(jnp.max(val_vec), (_L,))
        tie = val_vec == wv
        wi = jnp.broadcast_to(
            jnp.min(jnp.where(tie, idx_vec, jnp.int32(_BIG_I))), (_L,))
        tw = lax.shift_right_logical(wi * jnp.int32(26215), 25)

        def fld(k):
            return plsc.load_gather(
                cand_v, [tw, jnp.full((_L,), 2 * _L + k, jnp.int32)])
        wx1, wy1, wx2, wy2, wsc, wa = (fld(0), fld(1), fld(2), fld(3),
                                       fld(4), fld(5))

        out_v[it] = jnp.where(iot == 0, wx1,
                    jnp.where(iot == 1, wy1,
                    jnp.where(iot == 2, wx2,
                    jnp.where(iot == 3, wy2, wsc))))

        lv = jnp.clip(wi - base, 0, _PER - 1)
        owner = (wi >= base) & (wi < base + _PER)
        plsc.store_scatter(
            sw_v,
            [lax.shift_right_arithmetic(lv, 4), lax.bitwise_and(lv, 15)],
            jnp.full((_L,), ninf, jnp.float32),
            mask=owner & (iot == 0))

        def sweep(r, carry):
            bv, bi = carry
            x1 = x1_v[r]
            y1 = y1_v[r]
            x2 = x2_v[r]
            y2 = y2_v[r]
            ix1 = jnp.maximum(wx1, x1)
            iy1 = jnp.maximum(wy1, y1)
            ix2 = jnp.minimum(wx2, x2)
            iy2 = jnp.minimum(wy2, y2)
            inter = jnp.maximum(ix2 - ix1, 0.0) * jnp.maximum(iy2 - iy1, 0.0)
            ar = (x2 - x1) * (y2 - y1)
            union = wa + ar - inter + jnp.float32(1e-6)
            iou = inter / union
            idxv = iot + (base + r * _L)
            supp = iou > _NMS_THRESH
            sw = jnp.where(supp, ninf, sw_v[r])
            sw_v[r] = sw
            cond = sw > bv
            return jnp.where(cond, sw, bv), jnp.where(cond, idxv, bi)

        bv, bi = lax.fori_loop(0, _ROWS, sweep, (bv0, bi0))
        publish(bv, bi, _NS - roff)

    @pl.when((c == 0) & (s == 0))
    def _store():
        pltpu.sync_copy(out_v, out_hbm)


@jax.jit
def kernel(boxes, scores):
    pad = _NP - _N
    s_p = jnp.concatenate([scores, jnp.zeros((pad,), jnp.float32)])
    b_p = jnp.concatenate([boxes, jnp.zeros((pad, 4), jnp.float32)], axis=0)
    s3 = s_p.reshape(_NS, _ROWS, _L)
    x1 = b_p[:, 0].reshape(_NS, _ROWS, _L)
    y1 = b_p[:, 1].reshape(_NS, _ROWS, _L)
    x2 = b_p[:, 2].reshape(_NS, _ROWS, _L)
    y2 = b_p[:, 3].reshape(_NS, _ROWS, _L)

    cp = pltpu.CompilerParams()
    if "needs_layout_passes" in pltpu.CompilerParams.__dataclass_fields__:
        cp = dataclasses.replace(cp, needs_layout_passes=False)
    mesh = plsc.VectorSubcoreMesh(core_axis_name="c", subcore_axis_name="s")
    f = pl.kernel(
        _nms_body,
        out_type=jax.ShapeDtypeStruct((_MAX_DET, _L), jnp.float32),
        mesh=mesh,
        compiler_params=cp,
        scratch_types=[
            pltpu.VMEM((_ROWS, _L), jnp.float32),
            pltpu.VMEM((_ROWS, _L), jnp.float32),
            pltpu.VMEM((_ROWS, _L), jnp.float32),
            pltpu.VMEM((_ROWS, _L), jnp.float32),
            pltpu.VMEM((_ROWS, _L), jnp.float32),
            pltpu.VMEM((_ROWS, _L), jnp.float32),
            pltpu.VMEM((_ROWS, _L), jnp.float32),
            pltpu.VMEM((1, 128), jnp.float32),
            pltpu.VMEM_SHARED((2 * _NS, 128), jnp.float32),
            pltpu.VMEM((_NS, 128), jnp.float32),
            pltpu.VMEM((_MAX_DET, _L), jnp.float32),
        ],
    )
    out = f(s3, x1, y1, x2, y2)
    return out[:, :5]


# speedup vs baseline: 1.1331x; 1.1331x over previous
import dataclasses
import functools
import jax
import jax.numpy as jnp
from jax import lax
from jax.experimental import pallas as pl
from jax.experimental.pallas import tpu as pltpu
from jax.experimental.pallas import tpu_sc as plsc

_N = 20000
_L = 16
_NS = 16
_ROWS = 80
_PER = _ROWS * _L
_NP = _NS * _PER
_MAX_DET = 100
_SCORE_THRESH = 0.05
_NMS_THRESH = 0.5
_BIG_I = 1 << 30


def _nms_body(s_hbm, x1_hbm, y1_hbm, x2_hbm, y2_hbm, out_hbm,
              sw_v, so_v, x1_v, y1_v, x2_v, y2_v, ar_v,
              pub_v, shared_v, cand_v, out_v):
    c = lax.axis_index("c")
    s = lax.axis_index("s")
    base = s * _PER
    iot = lax.iota(jnp.int32, _L)
    ninf = jnp.float32(-jnp.inf)

    pltpu.sync_copy(s_hbm.at[s], so_v)
    pltpu.sync_copy(x1_hbm.at[s], x1_v)
    pltpu.sync_copy(y1_hbm.at[s], y1_v)
    pltpu.sync_copy(x2_hbm.at[s], x2_v)
    pltpu.sync_copy(y2_hbm.at[s], y2_v)

    def publish(bv, bi, buf_row):
        m_v = jnp.broadcast_to(jnp.max(bv), (_L,))
        li_v = jnp.broadcast_to(
            jnp.min(jnp.where(bv == m_v, bi, jnp.int32(_BIG_I))), (_L,))
        loc_v = li_v - base
        rowv = lax.shift_right_arithmetic(loc_v, 4)
        offv = lax.bitwise_and(loc_v, 15)
        gx1 = plsc.load_gather(x1_v, [rowv, offv])
        gy1 = plsc.load_gather(y1_v, [rowv, offv])
        gx2 = plsc.load_gather(x2_v, [rowv, offv])
        gy2 = plsc.load_gather(y2_v, [rowv, offv])
        gar = plsc.load_gather(ar_v, [rowv, offv])
        gso = plsc.load_gather(so_v, [rowv, offv])
        data = jnp.where(iot == 0, gx1,
               jnp.where(iot == 1, gy1,
               jnp.where(iot == 2, gx2,
               jnp.where(iot == 3, gy2,
               jnp.where(iot == 4, gso, gar)))))
        pub_v[0, 0:_L] = m_v
        pub_v[0, _L:2 * _L] = plsc.bitcast(li_v, jnp.float32)
        pub_v[0, 2 * _L:3 * _L] = data
        pltpu.sync_copy(pub_v, shared_v.at[pl.ds(buf_row + s, 1)])

    def prep(r, carry):
        bv, bi = carry
        v = so_v[r]
        sw = jnp.where(v > _SCORE_THRESH, v, ninf)
        sw_v[r] = sw
        ar_v[r] = (x2_v[r] - x1_v[r]) * (y2_v[r] - y1_v[r])
        cond = sw > bv
        idxv = iot + (base + r * _L)
        return jnp.where(cond, sw, bv), jnp.where(cond, idxv, bi)

    bv0 = jnp.full((_L,), ninf, jnp.float32)
    bi0 = jnp.full((_L,), base, jnp.int32)
    bv, bi = lax.fori_loop(0, _ROWS, prep, (bv0, bi0))
    publish(bv, bi, jnp.int32(0))

    zero_v = jnp.zeros((_L,), jnp.int32)
    one_v = jnp.full((_L,), _L, jnp.int32)

    @pl.loop(0, _MAX_DET)
    def _round(it):
        roff = lax.bitwise_and(it, 1) * _NS
        plsc.subcore_barrier()
        pltpu.sync_copy(shared_v.at[pl.ds(roff, _NS)], cand_v)

        val_vec = plsc.load_gather(cand_v, [iot, zero_v])
        idx_vec = plsc.bitcast(plsc.load_gather(cand_v, [iot, one_v]),
                               jnp.int32)
        wv = jnp.broadcast_to(jnp.max(val_vec), (_L,))
        tie = val_vec == wv
        wi = jnp.broadcast_to(
            jnp.min(jnp.where(tie, idx_vec, jnp.int32(_BIG_I))), (_L,))
        tw = lax.shift_right_logical(wi * jnp.int32(26215), 25)

        def fld(k):
            return plsc.load_gather(
                cand_v, [tw, jnp.full((_L,), 2 * _L + k, jnp.int32)])
        wx1, wy1, wx2, wy2, wsc, wa = (fld(0), fld(1), fld(2), fld(3),
                                       fld(4), fld(5))

        out_v[it] = jnp.where(iot == 0, wx1,
                    jnp.where(iot == 1, wy1,
                    jnp.where(iot == 2, wx2,
                    jnp.where(iot == 3, wy2, wsc))))

        lv = jnp.clip(wi - base, 0, _PER - 1)
        owner = (wi >= base) & (wi < base + _PER)
        plsc.store_scatter(
            sw_v,
            [lax.shift_right_arithmetic(lv, 4), lax.bitwise_and(lv, 15)],
            jnp.full((_L,), ninf, jnp.float32),
            mask=owner & (iot == 0))

        @plsc.parallel_loop(0, _ROWS, carry=(bv0, bi0))
        def sweep(r, carry):
            bv, bi = carry
            ix1 = jnp.maximum(wx1, x1_v[r])
            iy1 = jnp.maximum(wy1, y1_v[r])
            ix2 = jnp.minimum(wx2, x2_v[r])
            iy2 = jnp.minimum(wy2, y2_v[r])
            inter = jnp.maximum(ix2 - ix1, 0.0) * jnp.maximum(iy2 - iy1, 0.0)
            union = wa + ar_v[r] - inter + jnp.float32(1e-6)
            iou = inter / union
            idxv = iot + (base + r * _L)
            supp = iou > _NMS_THRESH
            sw = jnp.where(supp, ninf, sw_v[r])
            sw_v[r] = sw
            cond = sw > bv
            return jnp.where(cond, sw, bv), jnp.where(cond, idxv, bi)

        bv, bi = sweep
        publish(bv, bi, _NS - roff)

    @pl.when((c == 0) & (s == 0))
    def _store():
        pltpu.sync_copy(out_v, out_hbm)


@jax.jit
def kernel(boxes, scores):
    pad = _NP - _N
    s_p = jnp.concatenate([scores, jnp.zeros((pad,), jnp.float32)])
    b_p = jnp.concatenate([boxes, jnp.zeros((pad, 4), jnp.float32)], axis=0)
    s3 = s_p.reshape(_NS, _ROWS, _L)
    x1 = b_p[:, 0].reshape(_NS, _ROWS, _L)
    y1 = b_p[:, 1].reshape(_NS, _ROWS, _L)
    x2 = b_p[:, 2].reshape(_NS, _ROWS, _L)
    y2 = b_p[:, 3].reshape(_NS, _ROWS, _L)

    cp = pltpu.CompilerParams()
    if "needs_layout_passes" in pltpu.CompilerParams.__dataclass_fields__:
        cp = dataclasses.replace(cp, needs_layout_passes=False)
    mesh = plsc.VectorSubcoreMesh(core_axis_name="c", subcore_axis_name="s")
    f = pl.kernel(
        _nms_body,
        out_type=jax.ShapeDtypeStruct((_MAX_DET, _L), jnp.float32),
        mesh=mesh,
        compiler_params=cp,
        scratch_types=[
            pltpu.VMEM((_ROWS, _L), jnp.float32),
            pltpu.VMEM((_ROWS, _L), jnp.float32),
            pltpu.VMEM((_ROWS, _L), jnp.float32),
            pltpu.VMEM((_ROWS, _L), jnp.float32),
            pltpu.VMEM((_ROWS, _L), jnp.float32),
            pltpu.VMEM((_ROWS, _L), jnp.float32),
            pltpu.VMEM((_ROWS, _L), jnp.float32),
            pltpu.VMEM((1, 128), jnp.float32),
            pltpu.VMEM_SHARED((2 * _NS, 128), jnp.float32),
            pltpu.VMEM((_NS, 128), jnp.float32),
            pltpu.VMEM((_MAX_DET, _L), jnp.float32),
        ],
    )
    out = f(s3, x1, y1, x2, y2)
    return out[:, :5]


# speedup vs baseline: 1.1431x; 1.0088x over previous
import dataclasses
import functools
import jax
import jax.numpy as jnp
from jax import lax
from jax.experimental import pallas as pl
from jax.experimental.pallas import tpu as pltpu
from jax.experimental.pallas import tpu_sc as plsc

_N = 20000
_L = 16
_NS = 16
_ROWS = 79
_PER = _ROWS * _L
_NP = _NS * _PER
_MAX_DET = 100
_SCORE_THRESH = 0.05
_NMS_THRESH = 0.5
_BIG_I = 1 << 30


def _nms_body(s_hbm, x1_hbm, y1_hbm, x2_hbm, y2_hbm, out_hbm,
              sw_v, x1_v, y1_v, x2_v, y2_v, ar_v,
              pub_v, shared_v, cand_v, out_v, sem):
    c = lax.axis_index("c")
    s = lax.axis_index("s")
    base = s * _PER
    iot = lax.iota(jnp.int32, _L)
    ninf = jnp.float32(-jnp.inf)

    cps = [pltpu.async_copy(h.at[s], v, sem)
           for h, v in ((s_hbm, sw_v), (x1_hbm, x1_v), (y1_hbm, y1_v),
                        (x2_hbm, x2_v), (y2_hbm, y2_v))]
    for cp in cps:
        cp.wait()

    def publish(bv, bi, buf_row):
        m_v = jnp.broadcast_to(jnp.max(bv), (_L,))
        li_v = jnp.broadcast_to(
            jnp.min(jnp.where(bv == m_v, bi, jnp.int32(_BIG_I))), (_L,))
        loc_v = li_v - base
        rowv = lax.shift_right_arithmetic(loc_v, 4)
        offv = lax.bitwise_and(loc_v, 15)
        gx1 = plsc.load_gather(x1_v, [rowv, offv])
        gy1 = plsc.load_gather(y1_v, [rowv, offv])
        gx2 = plsc.load_gather(x2_v, [rowv, offv])
        gy2 = plsc.load_gather(y2_v, [rowv, offv])
        data = jnp.where(iot == 0, gx1,
               jnp.where(iot == 1, gy1,
               jnp.where(iot == 2, gx2, gy2)))
        pub_v[0, 0:_L] = m_v
        pub_v[0, _L:2 * _L] = plsc.bitcast(li_v, jnp.float32)
        pub_v[0, 2 * _L:3 * _L] = data
        pltpu.sync_copy(pub_v, shared_v.at[pl.ds(buf_row + s, 1)])

    def prep(r, carry):
        bv, bi = carry
        v = sw_v[r]
        sw = jnp.where(v > _SCORE_THRESH, v, ninf)
        sw_v[r] = sw
        ar_v[r] = (x2_v[r] - x1_v[r]) * (y2_v[r] - y1_v[r])
        cond = sw > bv
        idxv = iot + (base + r * _L)
        return jnp.where(cond, sw, bv), jnp.where(cond, idxv, bi)

    bv0 = jnp.full((_L,), ninf, jnp.float32)
    bi0 = jnp.full((_L,), base, jnp.int32)
    bv, bi = lax.fori_loop(0, _ROWS, prep, (bv0, bi0))
    publish(bv, bi, jnp.int32(0))

    zero_v = jnp.zeros((_L,), jnp.int32)
    one_v = jnp.full((_L,), _L, jnp.int32)

    @pl.loop(0, _MAX_DET)
    def _round(it):
        roff = lax.bitwise_and(it, 1) * _NS
        plsc.subcore_barrier()
        pltpu.sync_copy(shared_v.at[pl.ds(roff, _NS)], cand_v)

        val_vec = plsc.load_gather(cand_v, [iot, zero_v])
        idx_vec = plsc.bitcast(plsc.load_gather(cand_v, [iot, one_v]),
                               jnp.int32)
        wv = jnp.broadcast_to(jnp.max(val_vec), (_L,))
        tie = val_vec == wv
        wi = jnp.broadcast_to(
            jnp.min(jnp.where(tie, idx_vec, jnp.int32(_BIG_I))), (_L,))
        tw = lax.shift_right_logical(wi * jnp.int32(26215), 25)

        def fld(k):
            return plsc.load_gather(
                cand_v, [tw, jnp.full((_L,), 2 * _L + k, jnp.int32)])
        wx1, wy1, wx2, wy2 = fld(0), fld(1), fld(2), fld(3)
        wa = (wx2 - wx1) * (wy2 - wy1)

        out_v[it] = jnp.where(iot == 0, wx1,
                    jnp.where(iot == 1, wy1,
                    jnp.where(iot == 2, wx2,
                    jnp.where(iot == 3, wy2, wv))))

        lv = jnp.clip(wi - base, 0, _PER - 1)
        owner = (wi >= base) & (wi < base + _PER)
        plsc.store_scatter(
            sw_v,
            [lax.shift_right_arithmetic(lv, 4), lax.bitwise_and(lv, 15)],
            jnp.full((_L,), ninf, jnp.float32),
            mask=owner & (iot == 0))

        def sweep(r, carry):
            bv, bi = carry
            ix1 = jnp.maximum(wx1, x1_v[r])
            iy1 = jnp.maximum(wy1, y1_v[r])
            ix2 = jnp.minimum(wx2, x2_v[r])
            iy2 = jnp.minimum(wy2, y2_v[r])
            inter = jnp.maximum(ix2 - ix1, 0.0) * jnp.maximum(iy2 - iy1, 0.0)
            union = wa + ar_v[r] - inter + jnp.float32(1e-6)
            iou = inter / union
            idxv = iot + (base + r * _L)
            supp = iou > _NMS_THRESH
            sw = jnp.where(supp, ninf, sw_v[r])
            sw_v[r] = sw
            cond = sw > bv
            return jnp.where(cond, sw, bv), jnp.where(cond, idxv, bi)

        bv, bi = lax.fori_loop(0, _ROWS, sweep, (bv0, bi0))
        publish(bv, bi, _NS - roff)

    @pl.when((c == 0) & (s == 0))
    def _store():
        pltpu.sync_copy(out_v, out_hbm)


@jax.jit
def kernel(boxes, scores):
    pad = _NP - _N
    s_p = jnp.concatenate([scores, jnp.zeros((pad,), jnp.float32)])
    b_p = jnp.concatenate([boxes, jnp.zeros((pad, 4), jnp.float32)], axis=0)
    s3 = s_p.reshape(_NS, _ROWS, _L)
    x1 = b_p[:, 0].reshape(_NS, _ROWS, _L)
    y1 = b_p[:, 1].reshape(_NS, _ROWS, _L)
    x2 = b_p[:, 2].reshape(_NS, _ROWS, _L)
    y2 = b_p[:, 3].reshape(_NS, _ROWS, _L)

    cp = pltpu.CompilerParams()
    if "needs_layout_passes" in pltpu.CompilerParams.__dataclass_fields__:
        cp = dataclasses.replace(cp, needs_layout_passes=False)
    mesh = plsc.VectorSubcoreMesh(core_axis_name="c", subcore_axis_name="s")
    f = pl.kernel(
        _nms_body,
        out_type=jax.ShapeDtypeStruct((_MAX_DET, _L), jnp.float32),
        mesh=mesh,
        compiler_params=cp,
        scratch_types=[
            pltpu.VMEM((_ROWS, _L), jnp.float32),
            pltpu.VMEM((_ROWS, _L), jnp.float32),
            pltpu.VMEM((_ROWS, _L), jnp.float32),
            pltpu.VMEM((_ROWS, _L), jnp.float32),
            pltpu.VMEM((_ROWS, _L), jnp.float32),
            pltpu.VMEM((_ROWS, _L), jnp.float32),
            pltpu.VMEM((1, 128), jnp.float32),
            pltpu.VMEM_SHARED((2 * _NS, 128), jnp.float32),
            pltpu.VMEM((_NS, 128), jnp.float32),
            pltpu.VMEM((_MAX_DET, _L), jnp.float32),
            pltpu.SemaphoreType.DMA,
        ],
    )
    out = f(s3, x1, y1, x2, y2)
    res = out[:, :5]
    score = jnp.where(jnp.isneginf(res[:, 4:5]), scores[0], res[:, 4:5])
    return jnp.concatenate([res[:, :4], score], axis=1)
